# baseline (speedup 1.0000x reference)
import functools

import jax
import jax.numpy as jnp
from jax import lax
from jax.experimental import pallas as pl
from jax.experimental.pallas import tpu as pltpu
from jax.experimental.pallas import tpu_sc as plsc

_EMBED = 64
_NC = 2
_NS = 16
_NW = _NC * _NS
_NBUF = 8


@functools.partial(jax.jit, static_argnames=("batch", "hist"))
def _gather_rows(idx, table, *, batch, hist):
    bat_w = batch // _NW
    per_w = bat_w * hist
    assert bat_w > _NBUF
    mesh = plsc.VectorSubcoreMesh(core_axis_name="c", subcore_axis_name="s")

    @functools.partial(
        pl.kernel,
        out_type=jax.ShapeDtypeStruct((batch, hist, _EMBED), jnp.float32),
        mesh=mesh,
        scratch_types=[
            pltpu.VMEM((bat_w, hist), jnp.int32),
            pltpu.VMEM((_NBUF, hist, _EMBED), jnp.float32),
            [pltpu.SemaphoreType.DMA] * _NBUF,
            [pltpu.SemaphoreType.DMA] * _NBUF,
        ],
        compiler_params=pltpu.CompilerParams(use_tc_tiling_on_sc=False),
    )
    def body(idx_hbm, table_hbm, out_hbm, idx_v, rows, sem_g, sem_w):
        wid = lax.axis_index("s") * _NC + lax.axis_index("c")
        pltpu.sync_copy(idx_hbm.at[pl.ds(wid * bat_w, bat_w)], idx_v)

        def gather(g):
            b = g % _NBUF
            return pltpu.make_async_copy(
                table_hbm.at[idx_v.at[g]],
                rows.at[b], sem_g[b])

        def write(g):
            b = g % _NBUF
            return pltpu.make_async_copy(
                rows.at[b], out_hbm.at[wid * bat_w + g], sem_w[b])

        for g in range(_NBUF):
            gather(g).start()
        for g in range(bat_w):
            gather(g).wait()
            write(g).start()
            if g >= 1 and g - 1 + _NBUF < bat_w:
                write(g - 1).wait()
                gather(g - 1 + _NBUF).start()
        for g in range(bat_w - _NBUF, bat_w):
            write(g).wait()

    return body(idx, table)


def kernel(input, table):
    batch, hist = input.shape
    return _gather_rows(input.astype(jnp.int32), table, batch=batch, hist=hist)
